# baseline (speedup 1.0000x reference)
import jax
import jax.numpy as jnp
from jax import lax
from jax.experimental import pallas as pl
from jax.experimental.pallas import tpu as pltpu
from jax.experimental.pallas import tpu_sc as plsc

_N = 10000
_D = 128
_CHUNK = 128
_HALF = 5000
_HP = 5120
_RPT = _HP // 16
_NCH_T = 8
_NCH_G = 160
_SUB = 40


def _make_sc_scatter():
    mesh = plsc.VectorSubcoreMesh(core_axis_name="c", subcore_axis_name="s")

    out_type = (
        jax.ShapeDtypeStruct((2, _HP, _D), jnp.float32),
        jax.ShapeDtypeStruct((2, _HP, _D), jnp.float32),
    )
    scratch = (
        pltpu.VMEM((_SUB, _CHUNK), jnp.int32),
        pltpu.VMEM((_SUB, _CHUNK), jnp.int32),
        pltpu.VMEM((_CHUNK, _D), jnp.float32),
        pltpu.VMEM((_CHUNK, _D), jnp.float32),
        pltpu.VMEM((_CHUNK, _D), jnp.float32),
        pltpu.VMEM((_CHUNK, _D), jnp.float32),
        pltpu.VMEM_SHARED((_HP, _D), jnp.float32),
        pltpu.SemaphoreType.DMA, pltpu.SemaphoreType.DMA,
        pltpu.SemaphoreType.DMA, pltpu.SemaphoreType.DMA,
    )

    def body(vh_hbm, vs_hbm, st_hbm, dt_hbm, sg_hbm, dg_hbm, zrow_hbm,
             tt_hbm, tg_hbm,
             srcv, dstv, b0, b1, b2, b3, acc, g0, g1, g2, g3):
        cid = lax.axis_index("c")
        sid = lax.axis_index("s")
        base = sid * _RPT
        bufs = (b0, b1, b2, b3)
        gsem = (g0, g1, g2, g3)

        def phase(v_hbm, s_hbm, d_hbm, o_hbm, nsub, nch):
            pltpu.sync_copy(zrow_hbm.at[pl.ds(base, _RPT)],
                            acc.at[pl.ds(base, _RPT)])
            for sp in range(nsub):
                pltpu.sync_copy(s_hbm.at[sid, pl.ds(sp * nch, nch)],
                                srcv.at[pl.ds(0, nch)])
                pltpu.sync_copy(d_hbm.at[cid, sid, pl.ds(sp * nch, nch)],
                                dstv.at[pl.ds(0, nch)])
                pltpu.async_copy(v_hbm.at[srcv.at[0]], bufs[0], gsem[0])
                pltpu.async_copy(v_hbm.at[srcv.at[1]], bufs[1], gsem[1])
                pltpu.async_copy(v_hbm.at[srcv.at[2]], bufs[2], gsem[2])
                if sp == 0:
                    plsc.subcore_barrier()

                @pl.loop(0, nch // 4)
                def _(i):
                    for b in range(4):
                        c = 4 * i + b
                        nb = (b + 3) % 4
                        pltpu.make_async_copy(v_hbm.at[srcv.at[c]],
                                              bufs[b], gsem[b]).wait()

                        @pl.when(c + 3 < nch)
                        def _():
                            pltpu.async_copy(v_hbm.at[srcv.at[c + 3]],
                                             bufs[nb], gsem[nb])

                        pltpu.sync_copy(bufs[b], acc.at[dstv.at[c]],
                                        add=True)

            plsc.subcore_barrier()
            pltpu.sync_copy(acc.at[pl.ds(base, _RPT)],
                            o_hbm.at[cid, pl.ds(base, _RPT)])

        phase(vh_hbm, st_hbm, dt_hbm, tt_hbm, 1, _NCH_T)
        phase(vs_hbm, sg_hbm, dg_hbm, tg_hbm, 4, _SUB)

    return pl.kernel(body, out_type=out_type, mesh=mesh,
                     scratch_types=scratch)


_sc_scatter = _make_sc_scatter()


_BLK = 1000
_GRID = _N // _BLK


def _colmask(shape):
    return lax.broadcasted_iota(jnp.int32, shape, 1) == 0


def _rownorm(h):
    return jnp.sqrt(jnp.sum(h * h, axis=-1, keepdims=True))


def _sphere_exp(h):
    n = _rownorm(h)
    ns = jnp.maximum(n, 1e-8)
    return jnp.where(_colmask(h.shape), jnp.cos(n), jnp.sin(ns) / ns * h)


_TWO_PI = 6.283185307179586
_D_LO = 4.4721360497002096e-04
_D_HI = 3.1411454399448931e+00


def _sphere_log_of_exp(h):
    n = _rownorm(h)
    m = n - _TWO_PI * jnp.floor(n / _TWO_PI)
    d = jnp.clip(jnp.minimum(m, _TWO_PI - m), _D_LO, _D_HI)
    coef = d / jnp.maximum(jnp.sin(d), 1e-8)
    ns = jnp.maximum(n, 1e-8)
    return jnp.where(_colmask(h.shape), 0.0, coef * (jnp.sin(ns) / ns) * h)


def _lorentz_exp(h):
    n = _rownorm(h)
    ns = jnp.maximum(n, 1e-8)
    cosh_n = 0.5 * (jnp.exp(n) + jnp.exp(-n))
    sinh_ns = 0.5 * (jnp.exp(ns) - jnp.exp(-ns))
    return jnp.where(_colmask(h.shape), cosh_n, sinh_ns / ns * h)


def _mm(a, w):
    return jnp.dot(a, w, preferred_element_type=jnp.float32)


def _agg(q_ref, c_ref):
    cnt = jnp.maximum(c_ref[0, :, 0:1], 1.0)
    return q_ref[0] / cnt


def _tc1_body(x_ref, we_ref, be_ref, wh_ref, ws_ref, xe_ref, vh_ref, v1s_ref):
    x = x_ref[...]
    xe_ref[...] = jnp.maximum(_mm(x, we_ref[...]) + be_ref[...], 0.0)
    zero0 = _colmask((x.shape[0], _D))
    vh_ref[...] = jnp.where(zero0, 1.0, _mm(x, wh_ref[...]))
    vs = jnp.where(zero0, 0.0, _mm(x, ws_ref[...]))
    v1s_ref[...] = jnp.where(zero0, 1.0, _sphere_log_of_exp(vs))


def _tc2_body(vh_ref, v1s_ref, tt_ref, tg_ref, u1h_ref, u1s_ref):
    u1h_ref[...] = vh_ref[...] + _agg(tt_ref, tt_ref)
    u1s_ref[...] = v1s_ref[...] + _agg(tg_ref, tg_ref)


def _tc3_body(u1h_ref, u1s_ref, qt_ref, tt_ref, qg_ref, tg_ref,
              wh1_ref, wh2_ref, ws1_ref, ws2_ref, xh_ref, xs_ref):
    zero0 = _colmask((_BLK, _D))
    ah = jnp.where(zero0, 0.0, u1h_ref[...] + _agg(qt_ref, tt_ref))
    u2h = jnp.where(zero0, 0.0, _mm(ah, wh1_ref[...]))
    xh_ref[...] = _lorentz_exp(jnp.where(zero0, 0.0, _mm(u2h, wh2_ref[...])))
    as_ = jnp.where(zero0, 0.0, u1s_ref[...] + _agg(qg_ref, tg_ref))
    u2s = jnp.where(zero0, 0.0, _mm(as_, ws1_ref[...]))
    xs_ref[...] = _sphere_exp(jnp.where(zero0, 0.0, _mm(u2s, ws2_ref[...])))


_row_spec = pl.BlockSpec((_BLK, _D), lambda i: (i, 0))
_w_spec = pl.BlockSpec((_D, _D), lambda i: (0, 0))
_b_spec = pl.BlockSpec((1, _D), lambda i: (0, 0))
_t_spec = pl.BlockSpec((1, _BLK, _D), lambda i: (i // 5, i % 5, 0))
_c_spec = pl.BlockSpec((1, _BLK, 16), lambda i: (i // 5, i % 5, 0))
_row_out = jax.ShapeDtypeStruct((_N, _D), jnp.float32)

_tc1 = pl.pallas_call(
    _tc1_body, grid=(_GRID,),
    in_specs=[_row_spec, _w_spec, _b_spec, _w_spec, _w_spec],
    out_specs=[_row_spec] * 3,
    out_shape=[_row_out] * 3,
)

_tc2 = pl.pallas_call(
    _tc2_body, grid=(_GRID,),
    in_specs=[_row_spec, _row_spec, _t_spec, _t_spec],
    out_specs=[_row_spec] * 2,
    out_shape=[_row_out] * 2,
)

_tc3 = pl.pallas_call(
    _tc3_body, grid=(_GRID,),
    in_specs=[_row_spec, _row_spec, _t_spec, _t_spec, _t_spec, _t_spec,
              _w_spec, _w_spec, _w_spec, _w_spec],
    out_specs=[_row_spec] * 2,
    out_shape=[_row_out] * 2,
)


def _prep_edges(edge_index, nch):
    e = edge_index.shape[1]
    pad = 16 * nch * _CHUNK - e
    lane = jnp.arange(pad, dtype=jnp.int32)
    src = jnp.concatenate([edge_index[0], lane % 64])
    dst = jnp.concatenate([edge_index[1], jnp.full((pad,), _N, jnp.int32)])
    local = dst[None, :] - jnp.array([[0], [_HALF]], jnp.int32)
    oob = (local < 0) | (local >= _HALF)
    dst2 = jnp.where(oob, _HALF + (dst[None, :] % 120), local)
    return (src.reshape(16, nch, _CHUNK),
            dst2.reshape(2, 16, nch, _CHUNK))


def kernel(x, edge_index_tree, edge_index_graph, W_E, b_E,
           W_H, W_S, W_H1, W_S1, W_H2, W_S2):
    src_t, dst_t = _prep_edges(edge_index_tree, _NCH_T)
    src_g, dst_g = _prep_edges(edge_index_graph, _NCH_G)
    zrow = jnp.zeros((_HP, _D), jnp.float32)

    x_e, v1h, v1s = _tc1(x, W_E, b_E.reshape(1, _D), W_H, W_S)

    tt, tg = _sc_scatter(v1h, v1s, src_t, dst_t, src_g, dst_g, zrow)
    u1h, u1s = _tc2(v1h, v1s, tt, tg)

    qt, qg = _sc_scatter(u1h, u1s, src_t, dst_t, src_g, dst_g, zrow)
    x_h, x_s = _tc3(u1h, u1s, qt, tt, qg, tg, W_H1, W_H2, W_S1, W_S2)

    return (x_e, x_h, x_s)


# speedup vs baseline: 1.0037x; 1.0037x over previous
import jax
import jax.numpy as jnp
from jax import lax
from jax.experimental import pallas as pl
from jax.experimental.pallas import tpu as pltpu
from jax.experimental.pallas import tpu_sc as plsc

_N = 10000
_D = 128
_CHUNK = 128
_HALF = 5000
_HP = 5120
_RPT = _HP // 16
_NCH_T = 8
_NCH_G = 160
_SUB = 40


def _make_sc_scatter():
    mesh = plsc.VectorSubcoreMesh(core_axis_name="c", subcore_axis_name="s")

    out_type = (
        jax.ShapeDtypeStruct((2, _HP, _D), jnp.float32),
        jax.ShapeDtypeStruct((2, _HP, _D), jnp.float32),
    )
    scratch = (
        pltpu.VMEM((_SUB, _CHUNK), jnp.int32),
        pltpu.VMEM((_SUB, _CHUNK), jnp.int32),
        pltpu.VMEM((_CHUNK, _D), jnp.float32),
        pltpu.VMEM((_CHUNK, _D), jnp.float32),
        pltpu.VMEM((_CHUNK, _D), jnp.float32),
        pltpu.VMEM((_CHUNK, _D), jnp.float32),
        pltpu.VMEM_SHARED((_HP, _D), jnp.float32),
        pltpu.SemaphoreType.DMA, pltpu.SemaphoreType.DMA,
        pltpu.SemaphoreType.DMA, pltpu.SemaphoreType.DMA,
    )

    def body(vh_hbm, vs_hbm, st_hbm, dt_hbm, sg_hbm, dg_hbm, zrow_hbm,
             tt_hbm, tg_hbm,
             srcv, dstv, b0, b1, b2, b3, acc, g0, g1, g2, g3):
        cid = lax.axis_index("c")
        sid = lax.axis_index("s")
        base = sid * _RPT
        bufs = (b0, b1, b2, b3)
        gsem = (g0, g1, g2, g3)

        def phase(v_hbm, s_hbm, d_hbm, o_hbm, nsub, nch):
            pltpu.sync_copy(zrow_hbm.at[pl.ds(base, _RPT)],
                            acc.at[pl.ds(base, _RPT)])
            for sp in range(nsub):
                pltpu.sync_copy(s_hbm.at[sid, pl.ds(sp * nch, nch)],
                                srcv.at[pl.ds(0, nch)])
                pltpu.sync_copy(d_hbm.at[cid, sid, pl.ds(sp * nch, nch)],
                                dstv.at[pl.ds(0, nch)])
                pltpu.async_copy(v_hbm.at[srcv.at[0]], bufs[0], gsem[0])
                pltpu.async_copy(v_hbm.at[srcv.at[1]], bufs[1], gsem[1])
                if sp == 0:
                    plsc.subcore_barrier()

                @pl.loop(0, nch // 4)
                def _(i):
                    for b in range(4):
                        c = 4 * i + b
                        nb = (b + 2) % 4
                        pltpu.make_async_copy(v_hbm.at[srcv.at[c]],
                                              bufs[b], gsem[b]).wait()

                        @pl.when(c + 2 < nch)
                        def _():
                            pltpu.async_copy(v_hbm.at[srcv.at[c + 2]],
                                             bufs[nb], gsem[nb])

                        pltpu.sync_copy(bufs[b], acc.at[dstv.at[c]],
                                        add=True)

            plsc.subcore_barrier()
            pltpu.sync_copy(acc.at[pl.ds(base, _RPT)],
                            o_hbm.at[cid, pl.ds(base, _RPT)])

        phase(vh_hbm, st_hbm, dt_hbm, tt_hbm, 1, _NCH_T)
        phase(vs_hbm, sg_hbm, dg_hbm, tg_hbm, 4, _SUB)

    return pl.kernel(body, out_type=out_type, mesh=mesh,
                     scratch_types=scratch)


_sc_scatter = _make_sc_scatter()


_BLK = 1000
_GRID = _N // _BLK


def _colmask(shape):
    return lax.broadcasted_iota(jnp.int32, shape, 1) == 0


def _rownorm(h):
    return jnp.sqrt(jnp.sum(h * h, axis=-1, keepdims=True))


def _sphere_exp(h):
    n = _rownorm(h)
    ns = jnp.maximum(n, 1e-8)
    return jnp.where(_colmask(h.shape), jnp.cos(n), jnp.sin(ns) / ns * h)


_TWO_PI = 6.283185307179586
_D_LO = 4.4721360497002096e-04
_D_HI = 3.1411454399448931e+00


def _sphere_log_of_exp(h):
    n = _rownorm(h)
    m = n - _TWO_PI * jnp.floor(n / _TWO_PI)
    d = jnp.clip(jnp.minimum(m, _TWO_PI - m), _D_LO, _D_HI)
    coef = d / jnp.maximum(jnp.sin(d), 1e-8)
    ns = jnp.maximum(n, 1e-8)
    return jnp.where(_colmask(h.shape), 0.0, coef * (jnp.sin(ns) / ns) * h)


def _lorentz_exp(h):
    n = _rownorm(h)
    ns = jnp.maximum(n, 1e-8)
    cosh_n = 0.5 * (jnp.exp(n) + jnp.exp(-n))
    sinh_ns = 0.5 * (jnp.exp(ns) - jnp.exp(-ns))
    return jnp.where(_colmask(h.shape), cosh_n, sinh_ns / ns * h)


def _mm(a, w):
    return jnp.dot(a, w, preferred_element_type=jnp.float32)


def _agg(q_ref, c_ref):
    cnt = jnp.maximum(c_ref[0, :, 0:1], 1.0)
    return q_ref[0] / cnt


def _tc1_body(x_ref, we_ref, be_ref, wh_ref, ws_ref, xe_ref, vh_ref, v1s_ref):
    x = x_ref[...]
    xe_ref[...] = jnp.maximum(_mm(x, we_ref[...]) + be_ref[...], 0.0)
    zero0 = _colmask((x.shape[0], _D))
    vh_ref[...] = jnp.where(zero0, 1.0, _mm(x, wh_ref[...]))
    vs = jnp.where(zero0, 0.0, _mm(x, ws_ref[...]))
    v1s_ref[...] = jnp.where(zero0, 1.0, _sphere_log_of_exp(vs))


def _tc2_body(vh_ref, v1s_ref, tt_ref, tg_ref, u1h_ref, u1s_ref):
    u1h_ref[...] = vh_ref[...] + _agg(tt_ref, tt_ref)
    u1s_ref[...] = v1s_ref[...] + _agg(tg_ref, tg_ref)


def _tc3_body(u1h_ref, u1s_ref, qt_ref, tt_ref, qg_ref, tg_ref,
              wh1_ref, wh2_ref, ws1_ref, ws2_ref, xh_ref, xs_ref):
    zero0 = _colmask((_BLK, _D))
    ah = jnp.where(zero0, 0.0, u1h_ref[...] + _agg(qt_ref, tt_ref))
    u2h = jnp.where(zero0, 0.0, _mm(ah, wh1_ref[...]))
    xh_ref[...] = _lorentz_exp(jnp.where(zero0, 0.0, _mm(u2h, wh2_ref[...])))
    as_ = jnp.where(zero0, 0.0, u1s_ref[...] + _agg(qg_ref, tg_ref))
    u2s = jnp.where(zero0, 0.0, _mm(as_, ws1_ref[...]))
    xs_ref[...] = _sphere_exp(jnp.where(zero0, 0.0, _mm(u2s, ws2_ref[...])))


_row_spec = pl.BlockSpec((_BLK, _D), lambda i: (i, 0))
_w_spec = pl.BlockSpec((_D, _D), lambda i: (0, 0))
_b_spec = pl.BlockSpec((1, _D), lambda i: (0, 0))
_t_spec = pl.BlockSpec((1, _BLK, _D), lambda i: (i // 5, i % 5, 0))
_c_spec = pl.BlockSpec((1, _BLK, 16), lambda i: (i // 5, i % 5, 0))
_row_out = jax.ShapeDtypeStruct((_N, _D), jnp.float32)

_tc1 = pl.pallas_call(
    _tc1_body, grid=(_GRID,),
    in_specs=[_row_spec, _w_spec, _b_spec, _w_spec, _w_spec],
    out_specs=[_row_spec] * 3,
    out_shape=[_row_out] * 3,
)

_tc2 = pl.pallas_call(
    _tc2_body, grid=(_GRID,),
    in_specs=[_row_spec, _row_spec, _t_spec, _t_spec],
    out_specs=[_row_spec] * 2,
    out_shape=[_row_out] * 2,
)

_tc3 = pl.pallas_call(
    _tc3_body, grid=(_GRID,),
    in_specs=[_row_spec, _row_spec, _t_spec, _t_spec, _t_spec, _t_spec,
              _w_spec, _w_spec, _w_spec, _w_spec],
    out_specs=[_row_spec] * 2,
    out_shape=[_row_out] * 2,
)


def _prep_edges(edge_index, nch):
    e = edge_index.shape[1]
    pad = 16 * nch * _CHUNK - e
    lane = jnp.arange(pad, dtype=jnp.int32)
    src = jnp.concatenate([edge_index[0], lane % 64])
    dst = jnp.concatenate([edge_index[1], jnp.full((pad,), _N, jnp.int32)])
    local = dst[None, :] - jnp.array([[0], [_HALF]], jnp.int32)
    oob = (local < 0) | (local >= _HALF)
    dst2 = jnp.where(oob, _HALF + (dst[None, :] % 120), local)
    return (src.reshape(16, nch, _CHUNK),
            dst2.reshape(2, 16, nch, _CHUNK))


def kernel(x, edge_index_tree, edge_index_graph, W_E, b_E,
           W_H, W_S, W_H1, W_S1, W_H2, W_S2):
    src_t, dst_t = _prep_edges(edge_index_tree, _NCH_T)
    src_g, dst_g = _prep_edges(edge_index_graph, _NCH_G)
    zrow = jnp.zeros((_HP, _D), jnp.float32)

    x_e, v1h, v1s = _tc1(x, W_E, b_E.reshape(1, _D), W_H, W_S)

    tt, tg = _sc_scatter(v1h, v1s, src_t, dst_t, src_g, dst_g, zrow)
    u1h, u1s = _tc2(v1h, v1s, tt, tg)

    qt, qg = _sc_scatter(u1h, u1s, src_t, dst_t, src_g, dst_g, zrow)
    x_h, x_s = _tc3(u1h, u1s, qt, tt, qg, tg, W_H1, W_H2, W_S1, W_S2)

    return (x_e, x_h, x_s)
